# baseline (speedup 1.0000x reference)
import jax
import jax.numpy as jnp
import numpy as np
from jax.experimental import pallas as pl
from jax.experimental.pallas import tpu as pltpu

N = 128
KAPPA = 0.276
NE = N * (N + 1) // 2
BB = 128
NB = 1024 // BB


def _build_consts():
    r, c = np.tril_indices(N)
    a0 = c >> 4
    a1 = (c >> 1) & 7
    site = (a0 * 8 + a1).astype(np.int32)
    masks = np.zeros((8, NE), np.float32)
    masks[0] = (c <= 111) & (c + 16 <= r)
    masks[1] = c >= 112
    masks[2] = (a1 <= 6) & (c + 2 <= r)
    masks[3] = a1 == 7
    idx = np.zeros((8, NE), np.int32)
    idx[0] = site
    return masks, idx


_MASKS, _IDX = _build_consts()


def _lroll(x, k):
    k = k % NE
    return jnp.concatenate([x[:, k:], x[:, :k]], axis=1)


def _shift_right_128(x):
    return jnp.concatenate(
        [jnp.zeros((BB, 128), jnp.float32), x[:, :NE - 128]], axis=1)


def _kloss_kernel(net_ref, u_ref, mask_ref, idx_ref, out_ref):
    step = pl.program_id(0)

    @pl.when(step == 0)
    def _():
        out_ref[...] = jnp.zeros((1, 1, 128), jnp.float32)

    x = net_ref[...]
    m16 = mask_ref[0:1, :]
    m112 = mask_ref[1:2, :]
    m2 = mask_ref[2:3, :]
    m14 = mask_ref[3:4, :]

    y16 = _lroll(x, 16)
    s0 = m16 * y16 + m112 * _shift_right_128(y16)
    s1 = m2 * _lroll(x, 2) + m14 * _lroll(x, -14)

    u4 = u_ref[...]
    u = jnp.concatenate(
        [u4[:, d, a0, :] for d in range(2) for a0 in range(8)], axis=1)

    cu = jnp.cos(u) * (-2.0 * KAPPA)
    packed = pltpu.bitcast(
        pltpu.pack_elementwise([cu[:, :64], cu[:, 64:]],
                               packed_dtype=jnp.bfloat16),
        jnp.int32)
    idx = jnp.broadcast_to(idx_ref[0:1, :], (BB, NE))
    g = jnp.take_along_axis(packed, idx, axis=1)
    cg0 = pltpu.unpack_elementwise(
        g, index=0, packed_dtype=jnp.bfloat16, unpacked_dtype=jnp.float32)
    cg1 = pltpu.unpack_elementwise(
        g, index=1, packed_dtype=jnp.bfloat16, unpacked_dtype=jnp.float32)

    val = x * (x + s0 * cg0 + s1 * cg1)
    out_ref[...] += jnp.full((1, 1, 128), jnp.sum(val), jnp.float32)

    @pl.when(step == NB - 1)
    def _():
        out_ref[...] = out_ref[...] * (1.0 / (1024.0 * 128.0))


@jax.jit
def kernel(net_out, U1):
    out = pl.pallas_call(
        _kloss_kernel,
        grid=(NB,),
        in_specs=[
            pl.BlockSpec((BB, NE), lambda i: (i, 0)),
            pl.BlockSpec((BB, 2, 8, 8), lambda i: (i, 0, 0, 0)),
            pl.BlockSpec((8, NE), lambda i: (0, 0)),
            pl.BlockSpec((8, NE), lambda i: (0, 0)),
        ],
        out_specs=pl.BlockSpec((1, 1, 128), lambda i: (0, 0, 0)),
        out_shape=jax.ShapeDtypeStruct((1, 1, 128), jnp.float32),
        compiler_params=pltpu.CompilerParams(
            dimension_semantics=("arbitrary",),
            vmem_limit_bytes=100 * 1024 * 1024,
        ),
    )(net_out, U1, jnp.asarray(_MASKS), jnp.asarray(_IDX))
    return out[0, 0, 0]


# speedup vs baseline: 3.3364x; 3.3364x over previous
import jax
import jax.numpy as jnp
import numpy as np
from jax.experimental import pallas as pl
from jax.experimental.pallas import tpu as pltpu

N = 128
KAPPA = 0.276
NE = N * (N + 1) // 2
BBL = 128
NB = 1024 // BBL


def _build_consts():
    r, c = np.tril_indices(N)
    a0 = c >> 4
    a1 = (c >> 1) & 7
    site = a0 * 8 + a1
    m16 = (c <= 111) & (c + 16 <= r)
    m112 = c >= 112
    m2 = (a1 <= 6) & (c + 2 <= r)
    m14 = a1 == 7
    j = np.arange(NE)
    ot = np.zeros((128, 4 * NE), np.float32)
    ot[site, j] = m16
    ot[site, NE + j] = m112
    ot[64 + site, 2 * NE + j] = m2
    ot[64 + site, 3 * NE + j] = m14
    return ot.astype(jnp.bfloat16)


_OT = _build_consts()


def _ssh(x, k, ne=NE):
    if k >= 0:
        return jnp.concatenate(
            [x[k:], jnp.zeros((k, BBL), jnp.float32)], axis=0)
    return jnp.concatenate(
        [jnp.zeros((-k, BBL), jnp.float32), x[:ne + k]], axis=0)


def _kloss_kernel(net_ref, u_ref, ot_ref, out_ref):
    step = pl.program_id(0)

    @pl.when(step == 0)
    def _():
        out_ref[...] = jnp.zeros((1, 1, 128), jnp.float32)

    x = net_ref[...]
    sh2 = _ssh(x, 2)
    p = jnp.concatenate(
        [(x * _ssh(x, 16)).astype(jnp.bfloat16),
         (x * _ssh(x, -112)).astype(jnp.bfloat16),
         (x * sh2).astype(jnp.bfloat16),
         (x * _ssh(sh2, -16)).astype(jnp.bfloat16)], axis=0)
    d = jnp.dot(ot_ref[...], p, preferred_element_type=jnp.float32)

    u4 = u_ref[...]
    u = jnp.concatenate(
        [u4[d_, a0] for d_ in range(2) for a0 in range(8)], axis=0)
    cu = jnp.cos(u) * (-2.0 * KAPPA)

    total = jnp.sum(x * x) + jnp.sum(d * cu)
    out_ref[...] += jnp.full((1, 1, 128), total, jnp.float32)

    @pl.when(step == NB - 1)
    def _():
        out_ref[...] = out_ref[...] * (1.0 / (1024.0 * 128.0))


@jax.jit
def kernel(net_out, U1):
    xt = net_out.T
    ut = jnp.transpose(U1, (1, 2, 3, 0))
    out = pl.pallas_call(
        _kloss_kernel,
        grid=(NB,),
        in_specs=[
            pl.BlockSpec((NE, BBL), lambda i: (0, i)),
            pl.BlockSpec((2, 8, 8, BBL), lambda i: (0, 0, 0, i)),
            pl.BlockSpec((128, 4 * NE), lambda i: (0, 0)),
        ],
        out_specs=pl.BlockSpec((1, 1, 128), lambda i: (0, 0, 0)),
        out_shape=jax.ShapeDtypeStruct((1, 1, 128), jnp.float32),
        compiler_params=pltpu.CompilerParams(
            dimension_semantics=("arbitrary",),
            vmem_limit_bytes=100 * 1024 * 1024,
        ),
    )(xt, ut, jnp.asarray(_OT))
    return out[0, 0, 0]
